# baseline (speedup 1.0000x reference)
import functools

import numpy as np
import jax
import jax.numpy as jnp
from jax import lax
from jax.experimental import pallas as pl
from jax.experimental.pallas import tpu as pltpu
from jax.experimental.pallas import tpu_sc as plsc

R = 64
N = 8192
L = 16
CHUNK = N // L
U = 4
JB = CHUNK // U
NW = 32
ROWS_PER_W = R // NW
BINS = 256
INT_MIN = np.int32(-(2 ** 31))


def _sort_body(x_hbm, out_hbm, *scratch):
    dataf = scratch[0:2]
    keys = ((scratch[2], scratch[3]), (scratch[4], scratch[5]))
    hist = scratch[6:8]
    offs = (scratch[8:8 + U], scratch[8 + U:8 + 2 * U])
    sems = scratch[8 + 2 * U:8 + 2 * U + 2]

    wid = lax.axis_index("s") * 2 + lax.axis_index("c")
    lane = lax.iota(jnp.int32, L)
    ones = jnp.ones((L,), jnp.int32)
    zeros = jnp.zeros((L,), jnp.int32)
    zvec = jnp.zeros((L,), jnp.int32)
    fifteen = jnp.full((L,), 15, jnp.int32)
    lane9 = lane * CHUNK + lane
    rows = (wid * ROWS_PER_W, wid * ROWS_PER_W + 1)
    SU = [(r, u) for r in range(2) for u in range(U)]

    cin = [pltpu.async_copy(x_hbm.at[rows[r]], dataf[r], sems[r])
           for r in range(2)]

    def zero_body(i, c):
        for r in range(2):
            for u in range(U):
                hist[r][pl.ds((u * BINS + i) * L, L)] = zeros
        return c

    lax.fori_loop(0, BINS, zero_body, 0)
    for c in cin:
        c.wait()

    for p in range(4):
        shift = 8 * p

        def c_src(r):
            return (keys[r][1], keys[r][0], keys[r][1], keys[r][0])[p]

        def c_dst(r):
            return (keys[r][0], keys[r][1], keys[r][0], dataf[r])[p]

        def hmask(ks, sh):
            if sh >= 4:
                t = [lax.shift_right_logical(k, sh - 4) for k in ks]
            else:
                t = [lax.shift_left(k, 4) for k in ks]
            return [t_ & 0xFF0 for t_ in t]

        if p == 0:
            wb = (JB // L).bit_length() - 1

            def hist_body(i, c):
                l_s = lax.shift_right_logical(i, wb)
                w_s = i & (JB // L - 1)
                bases = [l_s * CHUNK + u * JB + w_s * L for u in range(U)]
                vs = [dataf[r][pl.ds(bases[u], L)] for (r, u) in SU]
                bs = [plsc.bitcast(v, jnp.int32) for v in vs]
                ms = [lax.shift_right_arithmetic(b, 31) for b in bs]
                ms = [m | INT_MIN for m in ms]
                ks = [b ^ m for b, m in zip(bs, ms)]
                for (r, u), k in zip(SU, ks):
                    keys[r][1][pl.ds(bases[u] + l_s, L)] = k
                hs = hmask(ks, 0)
                hidxs = [h | l_s for h in hs]
                for (r, u), h in zip(SU, hidxs):
                    plsc.addupdate_scatter(
                        hist[r], [h + np.int32(u * BINS * L)], ones)
                return c

            lax.fori_loop(0, CHUNK // U, hist_body, 0)

        def scan_body(i, carry):
            vs = [[hist[r][pl.ds((u * BINS + i) * L, L)] for u in range(U)]
                  for r in range(2)]
            for r in range(2):
                for u in range(U):
                    hist[r][pl.ds((u * BINS + i) * L, L)] = zeros
            t01 = [(v[0] + v[1], v[2] + v[3]) for v in vs]
            ts = [a + b for a, b in t01]
            css = [plsc.cumsum(t) for t in ts]
            excls = [cs - t + cry for cs, t, cry in zip(css, ts, carry)]
            tops = [jnp.take(cs, fifteen) for cs in css]
            nxt = tuple(cry + top for cry, top in zip(carry, tops))
            for r in range(2):
                acc = excls[r]
                for u in range(U):
                    offs[r][u][pl.ds(i * L, L)] = acc
                    if u < U - 1:
                        acc = acc + vs[r][u]
            return nxt

        lax.fori_loop(0, BINS, scan_body, (zvec, zvec))

        def scat_body(j, c):
            idxs = [lane9 + (u * JB + j) for u in range(U)]
            ks = [plsc.load_gather(c_src(r), [idxs[u]]) for (r, u) in SU]
            hs = hmask(ks, shift)
            hidxs = [h | lane for h in hs]
            poss = [plsc.load_gather(offs[r][u], [h])
                    for (r, u), h in zip(SU, hidxs)]
            if p == 3:
                ms = [lax.shift_right_arithmetic(k, 31) for k in ks]
                ms = [(~m) | INT_MIN for m in ms]
                outs = [plsc.bitcast(k ^ m, jnp.float32)
                        for k, m in zip(ks, ms)]
                st_poss = poss
                h2 = None
            else:
                outs = ks
                lps = [lax.shift_right_logical(pos, 9) for pos in poss]
                st_poss = [pos + lp for pos, lp in zip(poss, lps)]
                ub = int(U).bit_length() - 1
                ups = [lax.shift_left(pos, 3 + ub) & ((U - 1) << 12)
                       for pos in poss]
                dps = hmask(ks, shift + 8)
                h1 = [a | b for a, b in zip(ups, lps)]
                h2 = [a | b for a, b in zip(h1, dps)]
            for (r, u), pos, o in zip(SU, st_poss, outs):
                plsc.store_scatter(c_dst(r), [pos], o)
            for (r, u), h in zip(SU, hidxs):
                plsc.addupdate_scatter(offs[r][u], [h], ones)
            if h2 is not None:
                for (r, u), h in zip(SU, h2):
                    plsc.addupdate_scatter(hist[r], [h], ones)
            return c

        lax.fori_loop(0, JB, scat_body, 0)

    cout = [pltpu.async_copy(dataf[r], out_hbm.at[rows[r]], sems[r])
            for r in range(2)]
    for c in cout:
        c.wait()


_sc_sort = functools.partial(
    pl.kernel,
    out_type=jax.ShapeDtypeStruct((R, N), jnp.float32),
    mesh=plsc.VectorSubcoreMesh(core_axis_name="c", subcore_axis_name="s"),
    compiler_params=pltpu.CompilerParams(needs_layout_passes=False),
    scratch_types=[
        pltpu.VMEM((N,), jnp.float32),
        pltpu.VMEM((N,), jnp.float32),
        pltpu.VMEM((N + L,), jnp.int32),
        pltpu.VMEM((N + L,), jnp.int32),
        pltpu.VMEM((N + L,), jnp.int32),
        pltpu.VMEM((N + L,), jnp.int32),
        pltpu.VMEM((U * BINS * L,), jnp.int32),
        pltpu.VMEM((U * BINS * L,), jnp.int32),
    ] + [pltpu.VMEM((BINS * L,), jnp.int32) for _ in range(2 * U)]
      + [pltpu.SemaphoreType.DMA, pltpu.SemaphoreType.DMA],
)(_sort_body)


@jax.jit
def kernel(x):
    return _sc_sort(x)


# speedup vs baseline: 1.0082x; 1.0082x over previous
import functools

import numpy as np
import jax
import jax.numpy as jnp
from jax import lax
from jax.experimental import pallas as pl
from jax.experimental.pallas import tpu as pltpu
from jax.experimental.pallas import tpu_sc as plsc

R = 64
N = 8192
L = 16
CHUNK = N // L
U = 4
JB = CHUNK // U
NW = 32
ROWS_PER_W = R // NW
BINS = 256
INT_MIN = np.int32(-(2 ** 31))


def _sort_body(x_hbm, out_hbm, *scratch):
    dataf = scratch[0:2]
    keys = ((scratch[2], scratch[3]), (scratch[4], scratch[5]))
    hist = scratch[6:8]
    offs = (scratch[8:8 + U], scratch[8 + U:8 + 2 * U])
    sems = scratch[8 + 2 * U:8 + 2 * U + 2]

    wid = lax.axis_index("s") * 2 + lax.axis_index("c")
    lane = lax.iota(jnp.int32, L)
    ones = jnp.ones((L,), jnp.int32)
    zeros = jnp.zeros((L,), jnp.int32)
    zvec = jnp.zeros((L,), jnp.int32)
    fifteen = jnp.full((L,), 15, jnp.int32)
    lane9 = lane * CHUNK + lane
    rows = (wid * ROWS_PER_W, wid * ROWS_PER_W + 1)
    SU = [(r, u) for r in range(2) for u in range(U)]

    cin = [pltpu.async_copy(x_hbm.at[rows[r]], dataf[r], sems[r])
           for r in range(2)]

    def zero_body(i, c):
        for r in range(2):
            for u in range(U):
                hist[r][pl.ds((u * BINS + i) * L, L)] = zeros
        return c

    lax.fori_loop(0, BINS, zero_body, 0)
    for c in cin:
        c.wait()

    for p in range(4):
        shift = 8 * p

        def c_src(r):
            return (keys[r][1], keys[r][0], keys[r][1], keys[r][0])[p]

        def c_dst(r):
            return (keys[r][0], keys[r][1], keys[r][0], dataf[r])[p]

        def hmask(ks, sh):
            if sh >= 4:
                t = [lax.shift_right_logical(k, sh - 4) for k in ks]
            else:
                t = [lax.shift_left(k, 4) for k in ks]
            return [t_ & 0xFF0 for t_ in t]

        if p == 0:
            wb = (JB // L).bit_length() - 1

            def hist_body(i, c):
                l_s = lax.shift_right_logical(i, wb)
                w_s = i & (JB // L - 1)
                bases = [l_s * CHUNK + u * JB + w_s * L for u in range(U)]
                vs = [dataf[r][pl.ds(bases[u], L)] for (r, u) in SU]
                bs = [plsc.bitcast(v, jnp.int32) for v in vs]
                ms = [lax.shift_right_arithmetic(b, 31) for b in bs]
                ms = [m | INT_MIN for m in ms]
                ks = [b ^ m for b, m in zip(bs, ms)]
                for (r, u), k in zip(SU, ks):
                    keys[r][1][pl.ds(bases[u] + l_s, L)] = k
                hs = hmask(ks, 0)
                hidxs = [h | l_s for h in hs]
                for (r, u), h in zip(SU, hidxs):
                    plsc.addupdate_scatter(
                        hist[r], [h + np.int32(u * BINS * L)], ones)
                return c

            lax.fori_loop(0, CHUNK // U, hist_body, 0)

        def scan_body(i, carry):
            vs = [[hist[r][pl.ds((u * BINS + i) * L, L)] for u in range(U)]
                  for r in range(2)]
            for r in range(2):
                for u in range(U):
                    hist[r][pl.ds((u * BINS + i) * L, L)] = zeros
            t01 = [(v[0] + v[1], v[2] + v[3]) for v in vs]
            ts = [a + b for a, b in t01]
            css = [plsc.cumsum(t) for t in ts]
            excls = [cs - t + cry for cs, t, cry in zip(css, ts, carry)]
            tops = [jnp.take(cs, fifteen) for cs in css]
            nxt = tuple(cry + top for cry, top in zip(carry, tops))
            for r in range(2):
                acc = excls[r]
                for u in range(U):
                    offs[r][u][pl.ds(i * L, L)] = acc
                    if u < U - 1:
                        acc = acc + vs[r][u]
            return nxt

        lax.fori_loop(0, BINS, scan_body, (zvec, zvec))

        def load_ks(j):
            idxs = [lane9 + (u * JB + j) for u in range(U)]
            return tuple(plsc.load_gather(c_src(r), [idxs[u]])
                         for (r, u) in SU)

        def scat_body(j, ks):
            hs = hmask(ks, shift)
            hidxs = [h | lane for h in hs]
            poss = [plsc.load_gather(offs[r][u], [h])
                    for (r, u), h in zip(SU, hidxs)]
            if p == 3:
                ms = [lax.shift_right_arithmetic(k, 31) for k in ks]
                ms = [(~m) | INT_MIN for m in ms]
                outs = [plsc.bitcast(k ^ m, jnp.float32)
                        for k, m in zip(ks, ms)]
                st_poss = poss
                h2 = None
            else:
                outs = ks
                lps = [lax.shift_right_logical(pos, 9) for pos in poss]
                st_poss = [pos + lp for pos, lp in zip(poss, lps)]
                ub = int(U).bit_length() - 1
                ups = [lax.shift_left(pos, 3 + ub) & ((U - 1) << 12)
                       for pos in poss]
                dps = hmask(ks, shift + 8)
                h1 = [a | b for a, b in zip(ups, lps)]
                h2 = [a | b for a, b in zip(h1, dps)]
            for (r, u), pos, o in zip(SU, st_poss, outs):
                plsc.store_scatter(c_dst(r), [pos], o)
            for (r, u), h in zip(SU, hidxs):
                plsc.addupdate_scatter(offs[r][u], [h], ones)
            if h2 is not None:
                for (r, u), h in zip(SU, h2):
                    plsc.addupdate_scatter(hist[r], [h], ones)
            return load_ks(j + 1)

        lax.fori_loop(0, JB, scat_body, load_ks(0))

    cout = [pltpu.async_copy(dataf[r], out_hbm.at[rows[r]], sems[r])
            for r in range(2)]
    for c in cout:
        c.wait()


_sc_sort = functools.partial(
    pl.kernel,
    out_type=jax.ShapeDtypeStruct((R, N), jnp.float32),
    mesh=plsc.VectorSubcoreMesh(core_axis_name="c", subcore_axis_name="s"),
    compiler_params=pltpu.CompilerParams(needs_layout_passes=False),
    scratch_types=[
        pltpu.VMEM((N,), jnp.float32),
        pltpu.VMEM((N,), jnp.float32),
        pltpu.VMEM((N + L,), jnp.int32),
        pltpu.VMEM((N + L,), jnp.int32),
        pltpu.VMEM((N + L,), jnp.int32),
        pltpu.VMEM((N + L,), jnp.int32),
        pltpu.VMEM((U * BINS * L,), jnp.int32),
        pltpu.VMEM((U * BINS * L,), jnp.int32),
    ] + [pltpu.VMEM((BINS * L,), jnp.int32) for _ in range(2 * U)]
      + [pltpu.SemaphoreType.DMA, pltpu.SemaphoreType.DMA],
)(_sort_body)


@jax.jit
def kernel(x):
    return _sc_sort(x)


# speedup vs baseline: 1.0195x; 1.0112x over previous
import functools

import numpy as np
import jax
import jax.numpy as jnp
from jax import lax
from jax.experimental import pallas as pl
from jax.experimental.pallas import tpu as pltpu
from jax.experimental.pallas import tpu_sc as plsc

R = 64
N = 8192
L = 16
CHUNK = N // L
U = 4
JB = CHUNK // U
NW = 32
ROWS_PER_W = R // NW
BINS = 256
INT_MIN = np.int32(-(2 ** 31))


def _sort_body(x_hbm, out_hbm, *scratch):
    dataf = scratch[0:2]
    keys = ((scratch[2], scratch[3]), (scratch[4], scratch[5]))
    hist = scratch[6:8]
    offs = (scratch[8:8 + U], scratch[8 + U:8 + 2 * U])
    sems = scratch[8 + 2 * U:8 + 2 * U + 2]

    wid = lax.axis_index("s") * 2 + lax.axis_index("c")
    lane = lax.iota(jnp.int32, L)
    ones = jnp.ones((L,), jnp.int32)
    zeros = jnp.zeros((L,), jnp.int32)
    zvec = jnp.zeros((L,), jnp.int32)
    fifteen = jnp.full((L,), 15, jnp.int32)
    lane9 = lane * CHUNK + lane
    rows = (wid * ROWS_PER_W, wid * ROWS_PER_W + 1)
    SU = [(r, u) for r in range(2) for u in range(U)]

    cin = [pltpu.async_copy(x_hbm.at[rows[r]], dataf[r], sems[r])
           for r in range(2)]

    def zero_body(i, c):
        for r in range(2):
            for u in range(U):
                hist[r][pl.ds((u * BINS + i) * L, L)] = zeros
        return c

    lax.fori_loop(0, BINS, zero_body, 0)
    for c in cin:
        c.wait()

    for p in range(4):
        shift = 8 * p

        def c_src(r):
            return (keys[r][1], keys[r][0], keys[r][1], keys[r][0])[p]

        def c_dst(r):
            return (keys[r][0], keys[r][1], keys[r][0], dataf[r])[p]

        def hmask(ks, sh):
            if sh >= 4:
                t = [lax.shift_right_logical(k, sh - 4) for k in ks]
            else:
                t = [lax.shift_left(k, 4) for k in ks]
            return [t_ & 0xFF0 for t_ in t]

        if p == 0:
            wb = (JB // L).bit_length() - 1

            def load_vs(i):
                i = jnp.minimum(i, CHUNK // U - 1)
                l_s = lax.shift_right_logical(i, wb)
                w_s = i & (JB // L - 1)
                bases = [l_s * CHUNK + u * JB + w_s * L for u in range(U)]
                return (tuple(dataf[r][pl.ds(bases[u], L)]
                              for (r, u) in SU), l_s, bases)

            def hist_body(i, carry):
                vs, l_s, bases = carry
                bs = [plsc.bitcast(v, jnp.int32) for v in vs]
                ms = [lax.shift_right_arithmetic(b, 31) for b in bs]
                ms = [m | INT_MIN for m in ms]
                ks = [b ^ m for b, m in zip(bs, ms)]
                for (r, u), k in zip(SU, ks):
                    keys[r][1][pl.ds(bases[u] + l_s, L)] = k
                hs = hmask(ks, 0)
                hidxs = [h | l_s for h in hs]
                for (r, u), h in zip(SU, hidxs):
                    plsc.addupdate_scatter(
                        hist[r], [h + np.int32(u * BINS * L)], ones)
                return load_vs(i + 1)

            lax.fori_loop(0, CHUNK // U, hist_body, load_vs(0))

        def scan_body(i, carry):
            vs = [[hist[r][pl.ds((u * BINS + i) * L, L)] for u in range(U)]
                  for r in range(2)]
            if p < 3:
                for r in range(2):
                    for u in range(U):
                        hist[r][pl.ds((u * BINS + i) * L, L)] = zeros
            t01 = [(v[0] + v[1], v[2] + v[3]) for v in vs]
            ts = [a + b for a, b in t01]
            css = [plsc.cumsum(t) for t in ts]
            excls = [cs - t + cry for cs, t, cry in zip(css, ts, carry)]
            tops = [jnp.take(cs, fifteen) for cs in css]
            nxt = tuple(cry + top for cry, top in zip(carry, tops))
            for r in range(2):
                acc = excls[r]
                for u in range(U):
                    offs[r][u][pl.ds(i * L, L)] = acc
                    if u < U - 1:
                        acc = acc + vs[r][u]
            return nxt

        lax.fori_loop(0, BINS, scan_body, (zvec, zvec))

        def load_ks(j):
            idxs = [lane9 + (u * JB + j) for u in range(U)]
            return tuple(plsc.load_gather(c_src(r), [idxs[u]])
                         for (r, u) in SU)

        def scat_body(j, ks):
            hs = hmask(ks, shift)
            hidxs = [h | lane for h in hs]
            poss = [plsc.load_gather(offs[r][u], [h])
                    for (r, u), h in zip(SU, hidxs)]
            if p == 3:
                ms = [lax.shift_right_arithmetic(k, 31) for k in ks]
                ms = [(~m) | INT_MIN for m in ms]
                outs = [plsc.bitcast(k ^ m, jnp.float32)
                        for k, m in zip(ks, ms)]
                st_poss = poss
                h2 = None
            else:
                outs = ks
                lps = [lax.shift_right_logical(pos, 9) for pos in poss]
                st_poss = [pos + lp for pos, lp in zip(poss, lps)]
                ub = int(U).bit_length() - 1
                ups = [lax.shift_left(pos, 3 + ub) & ((U - 1) << 12)
                       for pos in poss]
                dps = hmask(ks, shift + 8)
                h1 = [a | b for a, b in zip(ups, lps)]
                h2 = [a | b for a, b in zip(h1, dps)]
            for (r, u), pos, o in zip(SU, st_poss, outs):
                plsc.store_scatter(c_dst(r), [pos], o)
            for (r, u), h in zip(SU, hidxs):
                plsc.addupdate_scatter(offs[r][u], [h], ones)
            if h2 is not None:
                for (r, u), h in zip(SU, h2):
                    plsc.addupdate_scatter(hist[r], [h], ones)
            return load_ks(j + 1)

        lax.fori_loop(0, JB, scat_body, load_ks(0))

    cout = [pltpu.async_copy(dataf[r], out_hbm.at[rows[r]], sems[r])
            for r in range(2)]
    for c in cout:
        c.wait()


_sc_sort = functools.partial(
    pl.kernel,
    out_type=jax.ShapeDtypeStruct((R, N), jnp.float32),
    mesh=plsc.VectorSubcoreMesh(core_axis_name="c", subcore_axis_name="s"),
    compiler_params=pltpu.CompilerParams(needs_layout_passes=False),
    scratch_types=[
        pltpu.VMEM((N,), jnp.float32),
        pltpu.VMEM((N,), jnp.float32),
        pltpu.VMEM((N + L,), jnp.int32),
        pltpu.VMEM((N + L,), jnp.int32),
        pltpu.VMEM((N + L,), jnp.int32),
        pltpu.VMEM((N + L,), jnp.int32),
        pltpu.VMEM((U * BINS * L,), jnp.int32),
        pltpu.VMEM((U * BINS * L,), jnp.int32),
    ] + [pltpu.VMEM((BINS * L,), jnp.int32) for _ in range(2 * U)]
      + [pltpu.SemaphoreType.DMA, pltpu.SemaphoreType.DMA],
)(_sort_body)


@jax.jit
def kernel(x):
    return _sc_sort(x)


# speedup vs baseline: 1.0208x; 1.0013x over previous
import functools

import numpy as np
import jax
import jax.numpy as jnp
from jax import lax
from jax.experimental import pallas as pl
from jax.experimental.pallas import tpu as pltpu
from jax.experimental.pallas import tpu_sc as plsc

R = 64
N = 8192
L = 16
CHUNK = N // L
U = 4
JB = CHUNK // U
NW = 32
ROWS_PER_W = R // NW
BINS = 256
INT_MIN = np.int32(-(2 ** 31))


def _sort_body(x_hbm, out_hbm, *scratch):
    dataf = scratch[0:2]
    keys = ((scratch[2], scratch[3]), (scratch[4], scratch[5]))
    hist = scratch[6:8]
    offs = (scratch[8:8 + U], scratch[8 + U:8 + 2 * U])
    sems = scratch[8 + 2 * U:8 + 2 * U + 2]

    wid = lax.axis_index("s") * 2 + lax.axis_index("c")
    lane = lax.iota(jnp.int32, L)
    ones = jnp.ones((L,), jnp.int32)
    zeros = jnp.zeros((L,), jnp.int32)
    zvec = jnp.zeros((L,), jnp.int32)
    fifteen = jnp.full((L,), 15, jnp.int32)
    lane9 = lane * CHUNK + lane
    rows = (wid * ROWS_PER_W, wid * ROWS_PER_W + 1)
    SU = [(r, u) for r in range(2) for u in range(U)]

    cin = [pltpu.async_copy(x_hbm.at[rows[r]], dataf[r], sems[r])
           for r in range(2)]

    def zero_body(i, c):
        for r in range(2):
            for u in range(U):
                hist[r][pl.ds((u * BINS + i) * L, L)] = zeros
        return c

    lax.fori_loop(0, BINS, zero_body, 0)
    for c in cin:
        c.wait()

    for p in range(4):
        shift = 8 * p

        def c_src(r):
            return (keys[r][1], keys[r][0], keys[r][1], keys[r][0])[p]

        def c_dst(r):
            return (keys[r][0], keys[r][1], keys[r][0], dataf[r])[p]

        def hmask(ks, sh):
            if sh >= 4:
                t = [lax.shift_right_logical(k, sh - 4) for k in ks]
            else:
                t = [lax.shift_left(k, 4) for k in ks]
            return [t_ & 0xFF0 for t_ in t]

        if p == 0:
            wb = (JB // L).bit_length() - 1

            def load_vs(i):
                i = jnp.minimum(i, CHUNK // U - 1)
                l_s = lax.shift_right_logical(i, wb)
                w_s = i & (JB // L - 1)
                bases = [l_s * CHUNK + u * JB + w_s * L for u in range(U)]
                return (tuple(dataf[r][pl.ds(bases[u], L)]
                              for (r, u) in SU), l_s, bases)

            def hist_body(i, carry):
                vs, l_s, bases = carry
                bs = [plsc.bitcast(v, jnp.int32) for v in vs]
                ms = [lax.shift_right_arithmetic(b, 31) for b in bs]
                ms = [m | INT_MIN for m in ms]
                ks = [b ^ m for b, m in zip(bs, ms)]
                for (r, u), k in zip(SU, ks):
                    keys[r][1][pl.ds(bases[u] + l_s, L)] = k
                hs = hmask(ks, 0)
                hidxs = [h | l_s for h in hs]
                for (r, u), h in zip(SU, hidxs):
                    plsc.addupdate_scatter(
                        hist[r], [h + np.int32(u * BINS * L)], ones)
                return load_vs(i + 1)

            lax.fori_loop(0, CHUNK // U, hist_body, load_vs(0))

        def scan_body(i, carry):
            vs = [[hist[r][pl.ds((u * BINS + i) * L, L)] for u in range(U)]
                  for r in range(2)]
            if p < 3:
                for r in range(2):
                    for u in range(U):
                        hist[r][pl.ds((u * BINS + i) * L, L)] = zeros
            def tree_sum(xs):
                while len(xs) > 1:
                    xs = [a + b for a, b in zip(xs[::2], xs[1::2])]
                return xs[0]

            ts = [tree_sum(list(v)) for v in vs]
            css = [plsc.cumsum(t) for t in ts]
            excls = [cs - t + cry for cs, t, cry in zip(css, ts, carry)]
            tops = [jnp.take(cs, fifteen) for cs in css]
            nxt = tuple(cry + top for cry, top in zip(carry, tops))
            for r in range(2):
                acc = excls[r]
                for u in range(U):
                    offs[r][u][pl.ds(i * L, L)] = acc
                    if u < U - 1:
                        acc = acc + vs[r][u]
            return nxt

        lax.fori_loop(0, BINS, scan_body, (zvec, zvec))

        def load_ks(j):
            idxs = [lane9 + (u * JB + j) for u in range(U)]
            return tuple(plsc.load_gather(c_src(r), [idxs[u]])
                         for (r, u) in SU)

        def scat_body(j, ks):
            hs = hmask(ks, shift)
            hidxs = [h | lane for h in hs]
            poss = [plsc.load_gather(offs[r][u], [h])
                    for (r, u), h in zip(SU, hidxs)]
            if p == 3:
                ms = [lax.shift_right_arithmetic(k, 31) for k in ks]
                ms = [(~m) | INT_MIN for m in ms]
                outs = [plsc.bitcast(k ^ m, jnp.float32)
                        for k, m in zip(ks, ms)]
                st_poss = poss
                h2 = None
            else:
                outs = ks
                lps = [lax.shift_right_logical(pos, 9) for pos in poss]
                st_poss = [pos + lp for pos, lp in zip(poss, lps)]
                ub = int(U).bit_length() - 1
                ups = [lax.shift_left(pos, 3 + ub) & ((U - 1) << 12)
                       for pos in poss]
                dps = hmask(ks, shift + 8)
                h1 = [a | b for a, b in zip(ups, lps)]
                h2 = [a | b for a, b in zip(h1, dps)]
            for (r, u), pos, o in zip(SU, st_poss, outs):
                plsc.store_scatter(c_dst(r), [pos], o)
            for (r, u), h in zip(SU, hidxs):
                plsc.addupdate_scatter(offs[r][u], [h], ones)
            if h2 is not None:
                for (r, u), h in zip(SU, h2):
                    plsc.addupdate_scatter(hist[r], [h], ones)
            return load_ks(j + 1)

        lax.fori_loop(0, JB, scat_body, load_ks(0))

    cout = [pltpu.async_copy(dataf[r], out_hbm.at[rows[r]], sems[r])
            for r in range(2)]
    for c in cout:
        c.wait()


_sc_sort = functools.partial(
    pl.kernel,
    out_type=jax.ShapeDtypeStruct((R, N), jnp.float32),
    mesh=plsc.VectorSubcoreMesh(core_axis_name="c", subcore_axis_name="s"),
    compiler_params=pltpu.CompilerParams(needs_layout_passes=False),
    scratch_types=[
        pltpu.VMEM((N,), jnp.float32),
        pltpu.VMEM((N,), jnp.float32),
        pltpu.VMEM((N + L,), jnp.int32),
        pltpu.VMEM((N + L,), jnp.int32),
        pltpu.VMEM((N + L,), jnp.int32),
        pltpu.VMEM((N + L,), jnp.int32),
        pltpu.VMEM((U * BINS * L,), jnp.int32),
        pltpu.VMEM((U * BINS * L,), jnp.int32),
    ] + [pltpu.VMEM((BINS * L,), jnp.int32) for _ in range(2 * U)]
      + [pltpu.SemaphoreType.DMA, pltpu.SemaphoreType.DMA],
)(_sort_body)


@jax.jit
def kernel(x):
    return _sc_sort(x)


# speedup vs baseline: 1.0220x; 1.0011x over previous
import functools

import numpy as np
import jax
import jax.numpy as jnp
from jax import lax
from jax.experimental import pallas as pl
from jax.experimental.pallas import tpu as pltpu
from jax.experimental.pallas import tpu_sc as plsc

R = 64
N = 8192
L = 16
CHUNK = N // L
U = 4
JB = CHUNK // U
NW = 32
ROWS_PER_W = R // NW
BINS = 256
INT_MIN = np.int32(-(2 ** 31))


def _sort_body(x_hbm, out_hbm, *scratch):
    dataf = scratch[0:2]
    keys = ((scratch[2], scratch[3]), (scratch[4], scratch[5]))
    hist = scratch[6:8]
    offs = (scratch[8:8 + U], scratch[8 + U:8 + 2 * U])
    sems = scratch[8 + 2 * U:8 + 2 * U + 2]

    wid = lax.axis_index("s") * 2 + lax.axis_index("c")
    lane = lax.iota(jnp.int32, L)
    ones = jnp.ones((L,), jnp.int32)
    zeros = jnp.zeros((L,), jnp.int32)
    zvec = jnp.zeros((L,), jnp.int32)
    fifteen = jnp.full((L,), 15, jnp.int32)
    lane9 = lane * CHUNK + lane
    rows = (wid * ROWS_PER_W, wid * ROWS_PER_W + 1)
    SU = [(r, u) for r in range(2) for u in range(U)]

    cin = [pltpu.async_copy(x_hbm.at[rows[r]], dataf[r], sems[r])
           for r in range(2)]

    def zero_body(i, c):
        for r in range(2):
            for u in range(U):
                hist[r][pl.ds((u * BINS + i) * L, L)] = zeros
        return c

    lax.fori_loop(0, BINS, zero_body, 0)
    for c in cin:
        c.wait()

    for p in range(4):
        shift = 8 * p

        def c_src(r):
            return (keys[r][1], keys[r][0], keys[r][1], keys[r][0])[p]

        def c_dst(r):
            return (keys[r][0], keys[r][1], keys[r][0], dataf[r])[p]

        def hmask(ks, sh):
            if sh >= 4:
                t = [lax.shift_right_logical(k, sh - 4) for k in ks]
            else:
                t = [lax.shift_left(k, 4) for k in ks]
            return [t_ & 0xFF0 for t_ in t]

        if p == 0:
            wb = (JB // L).bit_length() - 1

            def load_vs(i):
                i = jnp.minimum(i, CHUNK // U - 1)
                l_s = lax.shift_right_logical(i, wb)
                w_s = i & (JB // L - 1)
                bases = [l_s * CHUNK + u * JB + w_s * L for u in range(U)]
                return (tuple(dataf[r][pl.ds(bases[u], L)]
                              for (r, u) in SU), l_s, bases)

            def hist_body(i, carry):
                vs, l_s, bases = carry
                bs = [plsc.bitcast(v, jnp.int32) for v in vs]
                ms = [lax.shift_right_arithmetic(b, 31) for b in bs]
                ms = [m | INT_MIN for m in ms]
                ks = [b ^ m for b, m in zip(bs, ms)]
                for (r, u), k in zip(SU, ks):
                    keys[r][1][pl.ds(bases[u] + l_s, L)] = k
                hs = hmask(ks, 0)
                hidxs = [h | l_s for h in hs]
                for (r, u), h in zip(SU, hidxs):
                    plsc.addupdate_scatter(
                        hist[r], [h + np.int32(u * BINS * L)], ones)
                return load_vs(i + 1)

            lax.fori_loop(0, CHUNK // U, hist_body, load_vs(0))

        def scan_body(i, carry):
            vs = [[hist[r][pl.ds((u * BINS + i) * L, L)] for u in range(U)]
                  for r in range(2)]
            if p < 3:
                for r in range(2):
                    for u in range(U):
                        hist[r][pl.ds((u * BINS + i) * L, L)] = zeros
            def tree_sum(xs):
                while len(xs) > 1:
                    xs = [a + b for a, b in zip(xs[::2], xs[1::2])]
                return xs[0]

            ts = [tree_sum(list(v)) for v in vs]
            css = [plsc.cumsum(t) for t in ts]
            excls = [cs - t + cry for cs, t, cry in zip(css, ts, carry)]
            tops = [jnp.take(cs, fifteen) for cs in css]
            nxt = tuple(cry + top for cry, top in zip(carry, tops))
            for r in range(2):
                acc = excls[r]
                for u in range(U):
                    offs[r][u][pl.ds(i * L, L)] = acc
                    if u < U - 1:
                        acc = acc + vs[r][u]
            return nxt

        lax.fori_loop(0, BINS, scan_body, (zvec, zvec))

        def load_ks(j):
            idxs = [lane9 + (u * JB + j) for u in range(U)]
            ks = tuple(plsc.load_gather(c_src(r), [idxs[u]])
                       for (r, u) in SU)
            hidxs = tuple(h | lane for h in hmask(ks, shift))
            return ks, hidxs

        def scat_body(j, carry):
            ks, hidxs = carry
            poss = [plsc.load_gather(offs[r][u], [h])
                    for (r, u), h in zip(SU, hidxs)]
            if p == 3:
                ms = [lax.shift_right_arithmetic(k, 31) for k in ks]
                ms = [(~m) | INT_MIN for m in ms]
                outs = [plsc.bitcast(k ^ m, jnp.float32)
                        for k, m in zip(ks, ms)]
                st_poss = poss
                h2 = None
            else:
                outs = ks
                lps = [lax.shift_right_logical(pos, 9) for pos in poss]
                st_poss = [pos + lp for pos, lp in zip(poss, lps)]
                ub = int(U).bit_length() - 1
                ups = [lax.shift_left(pos, 3 + ub) & ((U - 1) << 12)
                       for pos in poss]
                dps = hmask(ks, shift + 8)
                h1 = [a | b for a, b in zip(ups, lps)]
                h2 = [a | b for a, b in zip(h1, dps)]
            for (r, u), pos, o in zip(SU, st_poss, outs):
                plsc.store_scatter(c_dst(r), [pos], o)
            for (r, u), h in zip(SU, hidxs):
                plsc.addupdate_scatter(offs[r][u], [h], ones)
            if h2 is not None:
                for (r, u), h in zip(SU, h2):
                    plsc.addupdate_scatter(hist[r], [h], ones)
            return load_ks(j + 1)

        lax.fori_loop(0, JB, scat_body, load_ks(0))

    cout = [pltpu.async_copy(dataf[r], out_hbm.at[rows[r]], sems[r])
            for r in range(2)]
    for c in cout:
        c.wait()


_sc_sort = functools.partial(
    pl.kernel,
    out_type=jax.ShapeDtypeStruct((R, N), jnp.float32),
    mesh=plsc.VectorSubcoreMesh(core_axis_name="c", subcore_axis_name="s"),
    compiler_params=pltpu.CompilerParams(needs_layout_passes=False),
    scratch_types=[
        pltpu.VMEM((N,), jnp.float32),
        pltpu.VMEM((N,), jnp.float32),
        pltpu.VMEM((N + L,), jnp.int32),
        pltpu.VMEM((N + L,), jnp.int32),
        pltpu.VMEM((N + L,), jnp.int32),
        pltpu.VMEM((N + L,), jnp.int32),
        pltpu.VMEM((U * BINS * L,), jnp.int32),
        pltpu.VMEM((U * BINS * L,), jnp.int32),
    ] + [pltpu.VMEM((BINS * L,), jnp.int32) for _ in range(2 * U)]
      + [pltpu.SemaphoreType.DMA, pltpu.SemaphoreType.DMA],
)(_sort_body)


@jax.jit
def kernel(x):
    return _sc_sort(x)
